# baseline (speedup 1.0000x reference)
import functools

import jax
import jax.numpy as jnp
from jax import lax
from jax.experimental import pallas as pl
from jax.experimental.pallas import tpu as pltpu
from jax.experimental.pallas import tpu_sc as plsc

B = 4096
L = 200
D = 64
N = B * L
NW = 32
PER_W = N // NW
CHUNK = 256
NCHUNK = PER_W // CHUNK
NBUF = 4


def _gather_body(idx_hbm, table_hbm, out_hbm, idx_v, rows_v, gsem, wsem):
    wid = lax.axis_index("s") * 2 + lax.axis_index("c")
    base = wid * PER_W

    pltpu.sync_copy(idx_hbm.at[wid], idx_v)

    def gather(i, b):
        return pltpu.make_async_copy(
            table_hbm.at[idx_v.at[i]], rows_v.at[b], gsem.at[b])

    def write(i, b):
        return pltpu.make_async_copy(
            rows_v.at[b], out_hbm.at[pl.ds(base + i * CHUNK, CHUNK)],
            wsem.at[b])

    for b in range(NBUF):
        gather(b, b).start()

    def group(g, carry):
        for b in range(NBUF):
            i = g + b
            gather(i, b).wait()
            write(i, b).start()
        for b in range(NBUF):
            i = g + b
            nxt = i + NBUF

            @pl.when(nxt < NCHUNK)
            def _():
                write(i, b).wait()
                gather(nxt, b).start()

        return carry

    lax.fori_loop(0, NCHUNK // NBUF, lambda k, c: group(k * NBUF, c), 0)

    for b in range(NBUF):
        write(NCHUNK - NBUF + b, b).wait()


_gather = functools.partial(
    pl.kernel,
    out_type=jax.ShapeDtypeStruct((N, D), jnp.float32),
    mesh=plsc.VectorSubcoreMesh(core_axis_name="c", subcore_axis_name="s"),
    scratch_types=[
        pltpu.VMEM((NCHUNK, CHUNK), jnp.int32),
        pltpu.VMEM((NBUF, CHUNK, D), jnp.float32),
        pltpu.SemaphoreType.DMA((NBUF,)),
        pltpu.SemaphoreType.DMA((NBUF,)),
    ],
    compiler_params=pltpu.CompilerParams(use_tc_tiling_on_sc=False),
)(_gather_body)


@jax.jit
def kernel(inputs, table):
    idx = (inputs.reshape(B // 8, 8, L // 2, 2)
           .transpose(0, 2, 1, 3)
           .reshape(NW, NCHUNK, CHUNK))
    rows = _gather(idx, table)
    out4 = rows.reshape(B // 8, L // 2, 8, 128)
    return out4.transpose(0, 2, 1, 3).reshape(B, L * D)


# speedup vs baseline: 1.4034x; 1.4034x over previous
import functools

import jax
import jax.numpy as jnp
from jax import lax
from jax.experimental import pallas as pl
from jax.experimental.pallas import tpu as pltpu
from jax.experimental.pallas import tpu_sc as plsc

B = 4096
L = 200
D = 64
N = B * L
NW = 32
PER_W = N // NW
TPC = 20
CHUNK = 16 * TPC
CPT = (L // 2) // TPC
NCHUNK = PER_W // CHUNK
NBUF = 4


def _gather_body(idx_hbm, table_hbm, out_hbm, idx_raw, pidx, rows_v,
                 gsem, wsem):
    wid = lax.axis_index("s") * 2 + lax.axis_index("c")
    base = wid * PER_W

    pltpu.sync_copy(idx_hbm.at[wid], idx_raw)

    lane = lax.iota(jnp.int32, 16)
    v0 = (lane >> 1) * L + (lane & 1)

    def build(c, b):
        ii = c // CPT
        j0 = (c % CPT) * TPC
        off = (8 * L) * ii + 2 * j0
        dst = pidx.at[b]
        for q in range(TPC):
            vals = plsc.load_gather(idx_raw, [v0 + (off + 2 * q)])
            dst[pl.ds(q * 16, 16)] = vals

    def gather(b):
        return pltpu.make_async_copy(
            table_hbm.at[pidx.at[b]], rows_v.at[b], gsem.at[b])

    def write(i, b):
        return pltpu.make_async_copy(
            rows_v.at[b], out_hbm.at[pl.ds(base + i * CHUNK, CHUNK)],
            wsem.at[b])

    for b in range(NBUF):
        build(b, b)
        gather(b).start()

    def group(g, carry):
        for b in range(NBUF):
            i = g + b
            gather(b).wait()
            write(i, b).start()
        for b in range(NBUF):
            i = g + b
            nxt = i + NBUF

            @pl.when(nxt < NCHUNK)
            def _():
                write(i, b).wait()
                build(nxt, b)
                gather(b).start()

        return carry

    lax.fori_loop(0, NCHUNK // NBUF, lambda k, c: group(k * NBUF, c), 0)

    for b in range(NBUF):
        write(NCHUNK - NBUF + b, b).wait()


_gather = functools.partial(
    pl.kernel,
    out_type=jax.ShapeDtypeStruct((N, D), jnp.float32),
    mesh=plsc.VectorSubcoreMesh(core_axis_name="c", subcore_axis_name="s"),
    scratch_types=[
        pltpu.VMEM((PER_W,), jnp.int32),
        pltpu.VMEM((NBUF, CHUNK), jnp.int32),
        pltpu.VMEM((NBUF, CHUNK, D), jnp.float32),
        pltpu.SemaphoreType.DMA((NBUF,)),
        pltpu.SemaphoreType.DMA((NBUF,)),
    ],
    compiler_params=pltpu.CompilerParams(
        use_tc_tiling_on_sc=False, needs_layout_passes=False),
)(_gather_body)


@jax.jit
def kernel(inputs, table):
    idx = inputs.reshape(NW, PER_W)
    rows = _gather(idx, table)
    out4 = rows.reshape(B // 8, L // 2, 8, 2 * D)
    return out4.transpose(0, 2, 1, 3).reshape(B, L * D)
